# baseline (speedup 1.0000x reference)
import functools

import jax
import jax.numpy as jnp
from jax import lax
from jax.experimental import pallas as pl
from jax.experimental.pallas import tpu as pltpu
from jax.experimental.pallas import tpu_sc as plsc

N = 10000
NQ = 2504
QB = (0, 2504, 5008, 7512, 10000)
E = 320000
H = 200
HH = 128
NUM_STEPS = 6

NC = 2
NS = 16
K = 128
CT = ((E + K - 1) // K + 4 + NS - 1) // NS + 1
CMAX = NS * CT
ECAP = CMAX * K

ROWS_PER_TILE = 160
ZLAST = NQ - 15 * ROWS_PER_TILE
TAIL_FULL = 2504 - 15 * ROWS_PER_TILE
TAIL_LAST = 2488 - 15 * ROWS_PER_TILE
AGG_ROWS = NQ + 8
DUMMY = NQ
NBUF = 4


def _sc_subpass(j, c, s, vec, lane, m_hbm, codep_hbm, zeros_hbm,
                out_hbm, sidx_all, didx_all, bufs, agg_sh, gsems, ssems):
    start = jnp.sum(jnp.where(lane == 4 * j + 2 * c, vec, 0))
    end = jnp.sum(jnp.where(lane == 4 * j + 2 * c + 1, vec, 0))
    base = pl.multiple_of(
        jnp.sum(jnp.where(lane == 8 + 2 * j + c, vec, 0)), 8)
    n_w = jnp.maximum(end - start - s + (NS - 1), 0) // NS
    trips = (n_w + NBUF - 1) // NBUF
    row = (start + s) % NS
    cs = (start + s) // NS

    pltpu.sync_copy(codep_hbm.at[row], didx_all)

    def unpack(i, carry):
        col = cs + i // (K // 16)
        off = (i % (K // 16)) * 16
        v = didx_all[col, pl.ds(off, 16)]
        sidx_all[col, pl.ds(off, 16)] = v >> 12
        didx_all[col, pl.ds(off, 16)] = v & 0xFFF
        return carry

    lax.fori_loop(0, n_w * (K // 16), unpack, 0)

    @pl.when(s < 15)
    def _():
        pltpu.sync_copy(zeros_hbm,
                        agg_sh.at[pl.ds(s * ROWS_PER_TILE, ROWS_PER_TILE)])

    @pl.when(s == 15)
    def _():
        pltpu.sync_copy(zeros_hbm.at[pl.ds(0, ZLAST)],
                        agg_sh.at[pl.ds(15 * ROWS_PER_TILE, ZLAST)])

    plsc.subcore_barrier()

    def body(i, carry):
        for u in range(NBUF):
            @pl.when(NBUF * i + u < n_w)
            def _(u=u):
                col = cs + NBUF * i + u

                @pl.when(i > 0)
                def _():
                    pltpu.make_async_copy(
                        bufs[u], agg_sh.at[didx_all.at[col]],
                        ssems[u]).wait()

                pltpu.async_copy(m_hbm.at[sidx_all.at[col]], bufs[u],
                                 gsems[u])
        for u in range(NBUF):
            @pl.when(NBUF * i + u < n_w)
            def _(u=u):
                col = cs + NBUF * i + u
                pltpu.make_async_copy(m_hbm.at[sidx_all.at[col]], bufs[u],
                                      gsems[u]).wait()
                pltpu.async_copy(bufs[u], agg_sh.at[didx_all.at[col]],
                                 ssems[u], add=True)
        return carry

    lax.fori_loop(0, trips, body, 0)
    for u in range(NBUF):
        @pl.when(n_w > u)
        def _(u=u):
            pltpu.make_async_copy(bufs[u], agg_sh.at[didx_all.at[cs]],
                                  ssems[u]).wait()
    plsc.subcore_barrier()

    @pl.when(s < 15)
    def _():
        r0 = s * ROWS_PER_TILE
        pltpu.sync_copy(agg_sh.at[pl.ds(r0, ROWS_PER_TILE)],
                        out_hbm.at[pl.ds(base + r0, ROWS_PER_TILE)])

    r0 = 15 * ROWS_PER_TILE
    if j == 1:
        @pl.when((s == 15) & (c == 0))
        def _():
            pltpu.sync_copy(agg_sh.at[pl.ds(r0, TAIL_FULL)],
                            out_hbm.at[pl.ds(base + r0, TAIL_FULL)])

        @pl.when((s == 15) & (c == 1))
        def _():
            pltpu.sync_copy(agg_sh.at[pl.ds(r0, TAIL_LAST)],
                            out_hbm.at[pl.ds(base + r0, TAIL_LAST)])
    else:
        @pl.when(s == 15)
        def _():
            pltpu.sync_copy(agg_sh.at[pl.ds(r0, TAIL_FULL)],
                            out_hbm.at[pl.ds(base + r0, TAIL_FULL)])


def _sc_scatter_entry(ma_hbm, mb_hbm, codep_hbm, bounds_hbm,
                      zeros_hbm, outa_hbm, outb_hbm,
                      sidx_all, didx_all,
                      buf0, buf1, buf2, buf3,
                      bnd_v, agg_sh,
                      gsem0, gsem1, gsem2, gsem3,
                      ssem0, ssem1, ssem2, ssem3):
    c = lax.axis_index("c")
    s = lax.axis_index("s")
    pltpu.sync_copy(bounds_hbm, bnd_v)
    vec = bnd_v[...]
    lane = lax.iota(jnp.int32, 16)
    bufs = (buf0, buf1, buf2, buf3)
    gsems = (gsem0, gsem1, gsem2, gsem3)
    ssems = (ssem0, ssem1, ssem2, ssem3)
    for m_hbm, out_hbm in ((ma_hbm, outa_hbm), (mb_hbm, outb_hbm)):
        for j in (0, 1):
            _sc_subpass(j, c, s, vec, lane, m_hbm, codep_hbm,
                        zeros_hbm, out_hbm, sidx_all, didx_all, bufs,
                        agg_sh, gsems, ssems)


@functools.cache
def _sc_scatter_kernel():
    return pl.kernel(
        _sc_scatter_entry,
        out_type=[jax.ShapeDtypeStruct((N, HH), jnp.float32),
                  jax.ShapeDtypeStruct((N, HH), jnp.float32)],
        mesh=plsc.VectorSubcoreMesh(core_axis_name="c", subcore_axis_name="s"),
        compiler_params=pltpu.CompilerParams(needs_layout_passes=False),
        scratch_types=(
            [pltpu.VMEM((CT, K), jnp.int32)] * 2
            + [pltpu.VMEM((K, HH), jnp.float32)] * NBUF
            + [pltpu.VMEM((16,), jnp.int32),
               pltpu.VMEM_SHARED((AGG_ROWS, HH), jnp.float32)]
            + [pltpu.SemaphoreType.DMA] * (2 * NBUF)
        ),
    )


def _sc_scatter(ma, mb, codep, bounds, zeros_hbm):
    return _sc_scatter_kernel()(ma, mb, codep, bounds, zeros_hbm)


_BM = 1000
_GRID = N // _BM


def _mm_body(h_ref, wa_ref, wb_ref, oa_ref, ob_ref):
    h = h_ref[...]
    oa_ref[...] = jnp.dot(h, wa_ref[...], preferred_element_type=jnp.float32)
    ob_ref[...] = jnp.dot(h, wb_ref[...], preferred_element_type=jnp.float32)


def _mm(h, wa, wb):
    return pl.pallas_call(
        _mm_body,
        grid=(_GRID,),
        in_specs=[
            pl.BlockSpec((_BM, H), lambda i: (i, 0)),
            pl.BlockSpec((H, HH), lambda i: (0, 0)),
            pl.BlockSpec((H, HH), lambda i: (0, 0)),
        ],
        out_specs=[
            pl.BlockSpec((_BM, HH), lambda i: (i, 0)),
            pl.BlockSpec((_BM, HH), lambda i: (i, 0)),
        ],
        out_shape=[
            jax.ShapeDtypeStruct((N, HH), jnp.float32),
            jax.ShapeDtypeStruct((N, HH), jnp.float32),
        ],
    )(h, wa, wb)


def _gru_math(aa, ab, h_ref,
              wira, wirb, wiza, wizb, wina, winb, whr, whz, whn,
              bir, biz, bin_, bhr, bhz, bhn):
    agga = aa[...]
    aggb = ab[...]
    h = h_ref[...]
    i_r = (jnp.dot(agga, wira[...], preferred_element_type=jnp.float32)
           + jnp.dot(aggb, wirb[...], preferred_element_type=jnp.float32)
           + bir[...])
    i_z = (jnp.dot(agga, wiza[...], preferred_element_type=jnp.float32)
           + jnp.dot(aggb, wizb[...], preferred_element_type=jnp.float32)
           + biz[...])
    i_n = (jnp.dot(agga, wina[...], preferred_element_type=jnp.float32)
           + jnp.dot(aggb, winb[...], preferred_element_type=jnp.float32)
           + bin_[...])
    h_r = jnp.dot(h, whr[...], preferred_element_type=jnp.float32) + bhr[...]
    h_z = jnp.dot(h, whz[...], preferred_element_type=jnp.float32) + bhz[...]
    h_n = jnp.dot(h, whn[...], preferred_element_type=jnp.float32) + bhn[...]
    r = jax.nn.sigmoid(i_r + h_r)
    z = jax.nn.sigmoid(i_z + h_z)
    n = jnp.tanh(i_n + r * h_n)
    return (1.0 - z) * n + z * h


def _gru_step_body(aa, ab, h_ref,
                   wira, wirb, wiza, wizb, wina, winb, whr, whz, whn,
                   bir, biz, bin_, bhr, bhz, bhn, wna, wnb,
                   h_out, ma_out, mb_out):
    hnew = _gru_math(aa, ab, h_ref,
                     wira, wirb, wiza, wizb, wina, winb, whr, whz, whn,
                     bir, biz, bin_, bhr, bhz, bhn)
    h_out[...] = hnew
    ma_out[...] = jnp.dot(hnew, wna[...], preferred_element_type=jnp.float32)
    mb_out[...] = jnp.dot(hnew, wnb[...], preferred_element_type=jnp.float32)


def _gru_pool_body(aa, ab, h_ref,
                   wira, wirb, wiza, wizb, wina, winb, whr, whz, whn,
                   bir, biz, bin_, bhr, bhz, bhn, pool_out):
    hnew = _gru_math(aa, ab, h_ref,
                     wira, wirb, wiza, wizb, wina, winb, whr, whz, whn,
                     bir, biz, bin_, bhr, bhz, bhn)

    @pl.when(pl.program_id(0) == 0)
    def _():
        pool_out[...] = jnp.zeros_like(pool_out)

    pool_out[...] += jnp.sum(hnew, axis=0, keepdims=True)


def _aspec():
    return pl.BlockSpec((_BM, HH), lambda i: (i, 0))


def _wspec():
    return pl.BlockSpec((HH, H), lambda i: (0, 0))


def _hwspec():
    return pl.BlockSpec((H, H), lambda i: (0, 0))


def _bspec():
    return pl.BlockSpec((1, H), lambda i: (0, 0))


_GRU_IN_SPECS = [
    _aspec(), _aspec(),
    pl.BlockSpec((_BM, H), lambda i: (i, 0)),
    _wspec(), _wspec(), _wspec(), _wspec(), _wspec(), _wspec(),
    _hwspec(), _hwspec(), _hwspec(),
    _bspec(), _bspec(), _bspec(),
    _bspec(), _bspec(), _bspec(),
]


def _gru_step(aa, ab, h, gw, wna, wnb):
    return pl.pallas_call(
        _gru_step_body,
        grid=(_GRID,),
        in_specs=_GRU_IN_SPECS + [pl.BlockSpec((H, HH), lambda i: (0, 0))] * 2,
        out_specs=[
            pl.BlockSpec((_BM, H), lambda i: (i, 0)),
            pl.BlockSpec((_BM, HH), lambda i: (i, 0)),
            pl.BlockSpec((_BM, HH), lambda i: (i, 0)),
        ],
        out_shape=[
            jax.ShapeDtypeStruct((N, H), jnp.float32),
            jax.ShapeDtypeStruct((N, HH), jnp.float32),
            jax.ShapeDtypeStruct((N, HH), jnp.float32),
        ],
    )(aa, ab, h, *gw, wna, wnb)


def _gru_pool(aa, ab, h, gw):
    return pl.pallas_call(
        _gru_pool_body,
        grid=(_GRID,),
        in_specs=_GRU_IN_SPECS,
        out_specs=pl.BlockSpec((1, H), lambda i: (0, 0)),
        out_shape=jax.ShapeDtypeStruct((1, H), jnp.float32),
    )(aa, ab, h, *gw)


def _mlp_body(p_ref, w1, b1, w2, b2, w3, b3, wc, bc, y_ref):
    o = jnp.maximum(jnp.dot(p_ref[...], w1[...],
                            preferred_element_type=jnp.float32) + b1[...], 0.0)
    o = jnp.maximum(jnp.dot(o, w2[...],
                            preferred_element_type=jnp.float32) + b2[...], 0.0)
    o = jnp.maximum(jnp.dot(o, w3[...],
                            preferred_element_type=jnp.float32) + b3[...], 0.0)
    y_ref[...] = jax.nn.sigmoid(
        jnp.dot(o, wc[...], preferred_element_type=jnp.float32) + bc[...])


def _mlp(pooled, w1, b1, w2, b2, w3, b3, wc, bc):
    return pl.pallas_call(
        _mlp_body,
        out_shape=jax.ShapeDtypeStruct((1, 2), jnp.float32),
    )(pooled, w1, b1, w2, b2, w3, b3, wc, bc)


def kernel(x, edge_index, ggnn_weight, gru_w_ih, gru_w_hh, gru_b_ih, gru_b_hh,
           ef_l1_w, ef_l1_b, ef_f1_w, ef_f1_b, ef_f2_w, ef_f2_b, cls_w, cls_b):
    ei = edge_index.astype(jnp.int32)
    src = ei[0]
    dst = ei[1]
    qb = jnp.asarray(QB, jnp.int32)
    q = ((dst >= QB[1]).astype(jnp.int32) + (dst >= QB[2])
         + (dst >= QB[3]))
    code = (src << 12) | (dst - jnp.take(qb, q))
    skey = lax.sort((q << 26) | code, is_stable=False)
    cbound = jnp.searchsorted(
        skey, jnp.arange(1, 5, dtype=jnp.int32) << 26).astype(jnp.int32)
    scode = skey & ((1 << 26) - 1)
    cstart = jnp.concatenate([jnp.zeros((1,), jnp.int32), cbound[:3]])
    cnts = cbound - cstart
    tchunks = (cnts + K - 1) // K
    bstart = jnp.concatenate(
        [jnp.zeros((1,), jnp.int32), jnp.cumsum(tchunks)])
    slot = jnp.arange(ECAP, dtype=jnp.int32)
    qslot = ((slot >= bstart[1] * K).astype(jnp.int32)
             + (slot >= bstart[2] * K) + (slot >= bstart[3] * K))
    off = slot - jnp.take(bstart, qslot) * K
    sidx = jnp.take(cstart, qslot) + off
    codep = jnp.where(
        off < jnp.take(cnts, qslot),
        jnp.take(scode, jnp.minimum(sidx, E - 1)), DUMMY)
    codep = codep.reshape(CT, NS, K).swapaxes(0, 1)
    bounds = jnp.zeros((16,), jnp.int32)
    for _j in (0, 1):
        for _c in (0, 1):
            _q = 2 * _j + _c
            bounds = (bounds.at[4 * _j + 2 * _c].set(bstart[_q])
                      .at[4 * _j + 2 * _c + 1].set(bstart[_q + 1])
                      .at[8 + 2 * _j + _c].set(QB[_q]))
    zeros_hbm = jnp.zeros((ROWS_PER_TILE, HH), jnp.float32)

    h0 = jnp.pad(x, ((0, 0), (0, H - x.shape[1])))
    wg = jnp.pad(ggnn_weight, ((0, 0), (0, 0), (0, 2 * HH - H)))
    wga = wg[:, :, :HH]
    wgb = wg[:, :, HH:]
    wihT = jnp.pad(gru_w_ih.T, ((0, 2 * HH - H), (0, 0)))
    whhT = gru_w_hh.T
    gw = (
        wihT[:HH, 0:H], wihT[HH:, 0:H],
        wihT[:HH, H:2 * H], wihT[HH:, H:2 * H],
        wihT[:HH, 2 * H:], wihT[HH:, 2 * H:],
        whhT[:, 0:H], whhT[:, H:2 * H], whhT[:, 2 * H:3 * H],
        gru_b_ih[0:H].reshape(1, H), gru_b_ih[H:2 * H].reshape(1, H),
        gru_b_ih[2 * H:].reshape(1, H),
        gru_b_hh[0:H].reshape(1, H), gru_b_hh[H:2 * H].reshape(1, H),
        gru_b_hh[2 * H:].reshape(1, H),
    )

    h = h0
    ma, mb = _mm(h0, wga[0], wgb[0])
    for i in range(NUM_STEPS):
        agg_a, agg_b = _sc_scatter(ma, mb, codep, bounds, zeros_hbm)
        if i < NUM_STEPS - 1:
            h, ma, mb = _gru_step(agg_a, agg_b, h, gw, wga[i + 1], wgb[i + 1])
        else:
            pooled = _gru_pool(agg_a, agg_b, h, gw)

    return _mlp(pooled,
                ef_l1_w.T, ef_l1_b.reshape(1, -1),
                ef_f1_w.T, ef_f1_b.reshape(1, -1),
                ef_f2_w.T, ef_f2_b.reshape(1, -1),
                cls_w.T, cls_b.reshape(1, -1))


# speedup vs baseline: 1.0240x; 1.0240x over previous
import functools

import jax
import jax.numpy as jnp
from jax import lax
from jax.experimental import pallas as pl
from jax.experimental.pallas import tpu as pltpu
from jax.experimental.pallas import tpu_sc as plsc

N = 10000
NQ = 2504
QB = (0, 2504, 5008, 7512, 10000)
E = 320000
H = 200
HH = 128
NUM_STEPS = 6

NC = 2
NS = 16
K = 128
CT = ((E + K - 1) // K + 4 + NS - 1) // NS + 1
CMAX = NS * CT
ECAP = CMAX * K

ROWS_PER_TILE = 160
ZLAST = NQ - 15 * ROWS_PER_TILE
TAIL_FULL = 2504 - 15 * ROWS_PER_TILE
TAIL_LAST = 2488 - 15 * ROWS_PER_TILE
AGG_ROWS = NQ + 8
DUMMY = NQ
NBUF = 4


def _sc_subpass(j, c, s, vec, lane, m_hbm, codep_hbm, zeros_hbm,
                out_hbm, sidx_all, didx_all, bufs, agg_sh, gsems, ssems):
    start = jnp.sum(jnp.where(lane == 4 * j + 2 * c, vec, 0))
    end = jnp.sum(jnp.where(lane == 4 * j + 2 * c + 1, vec, 0))
    base = pl.multiple_of(
        jnp.sum(jnp.where(lane == 8 + 2 * j + c, vec, 0)), 8)
    n_w = jnp.maximum(end - start - s + (NS - 1), 0) // NS
    trips = (n_w + NBUF - 1) // NBUF
    row = (start + s) % NS
    cs = (start + s) // NS

    pltpu.sync_copy(codep_hbm.at[row], didx_all)

    def unpack(i, carry):
        col = cs + i // (K // 16)
        off = (i % (K // 16)) * 16
        v = didx_all[col, pl.ds(off, 16)]
        sidx_all[col, pl.ds(off, 16)] = v >> 12
        didx_all[col, pl.ds(off, 16)] = v & 0xFFF
        return carry

    lax.fori_loop(0, n_w * (K // 16), unpack, 0)

    @pl.when(s < 15)
    def _():
        pltpu.sync_copy(zeros_hbm,
                        agg_sh.at[pl.ds(s * ROWS_PER_TILE, ROWS_PER_TILE)])

    @pl.when(s == 15)
    def _():
        pltpu.sync_copy(zeros_hbm.at[pl.ds(0, ZLAST)],
                        agg_sh.at[pl.ds(15 * ROWS_PER_TILE, ZLAST)])

    plsc.subcore_barrier()

    def body(i, carry):
        for u in range(NBUF):
            @pl.when(NBUF * i + u < n_w)
            def _(u=u):
                col = cs + NBUF * i + u

                @pl.when(i > 0)
                def _():
                    pltpu.make_async_copy(
                        bufs[u], agg_sh.at[didx_all.at[col]],
                        ssems[u]).wait()

                pltpu.async_copy(m_hbm.at[sidx_all.at[col]], bufs[u],
                                 gsems[u])
        for u in range(NBUF):
            @pl.when(NBUF * i + u < n_w)
            def _(u=u):
                col = cs + NBUF * i + u
                pltpu.make_async_copy(m_hbm.at[sidx_all.at[col]], bufs[u],
                                      gsems[u]).wait()
                pltpu.async_copy(bufs[u], agg_sh.at[didx_all.at[col]],
                                 ssems[u], add=True)
        return carry

    lax.fori_loop(0, trips, body, 0)
    for u in range(NBUF):
        @pl.when(n_w > u)
        def _(u=u):
            pltpu.make_async_copy(bufs[u], agg_sh.at[didx_all.at[cs]],
                                  ssems[u]).wait()
    plsc.subcore_barrier()

    @pl.when(s < 15)
    def _():
        r0 = s * ROWS_PER_TILE
        pltpu.sync_copy(agg_sh.at[pl.ds(r0, ROWS_PER_TILE)],
                        out_hbm.at[pl.ds(base + r0, ROWS_PER_TILE)])

    r0 = 15 * ROWS_PER_TILE
    if j == 1:
        @pl.when((s == 15) & (c == 0))
        def _():
            pltpu.sync_copy(agg_sh.at[pl.ds(r0, TAIL_FULL)],
                            out_hbm.at[pl.ds(base + r0, TAIL_FULL)])

        @pl.when((s == 15) & (c == 1))
        def _():
            pltpu.sync_copy(agg_sh.at[pl.ds(r0, TAIL_LAST)],
                            out_hbm.at[pl.ds(base + r0, TAIL_LAST)])
    else:
        @pl.when(s == 15)
        def _():
            pltpu.sync_copy(agg_sh.at[pl.ds(r0, TAIL_FULL)],
                            out_hbm.at[pl.ds(base + r0, TAIL_FULL)])


def _sc_scatter_entry(ma_hbm, mb_hbm, codep_hbm, bounds_hbm,
                      zeros_hbm, outa_hbm, outb_hbm,
                      sidx_all, didx_all,
                      buf0, buf1, buf2, buf3,
                      bnd_v, agg_sh,
                      gsem0, gsem1, gsem2, gsem3,
                      ssem0, ssem1, ssem2, ssem3):
    c = lax.axis_index("c")
    s = lax.axis_index("s")
    pltpu.sync_copy(bounds_hbm, bnd_v)
    vec = bnd_v[...]
    lane = lax.iota(jnp.int32, 16)
    bufs = (buf0, buf1, buf2, buf3)
    gsems = (gsem0, gsem1, gsem2, gsem3)
    ssems = (ssem0, ssem1, ssem2, ssem3)
    for m_hbm, out_hbm in ((ma_hbm, outa_hbm), (mb_hbm, outb_hbm)):
        for j in (0, 1):
            _sc_subpass(j, c, s, vec, lane, m_hbm, codep_hbm,
                        zeros_hbm, out_hbm, sidx_all, didx_all, bufs,
                        agg_sh, gsems, ssems)


@functools.cache
def _sc_scatter_kernel():
    return pl.kernel(
        _sc_scatter_entry,
        out_type=[jax.ShapeDtypeStruct((N, HH), jnp.float32),
                  jax.ShapeDtypeStruct((N, HH), jnp.float32)],
        mesh=plsc.VectorSubcoreMesh(core_axis_name="c", subcore_axis_name="s"),
        compiler_params=pltpu.CompilerParams(needs_layout_passes=False),
        scratch_types=(
            [pltpu.VMEM((CT, K), jnp.int32)] * 2
            + [pltpu.VMEM((K, HH), jnp.float32)] * NBUF
            + [pltpu.VMEM((16,), jnp.int32),
               pltpu.VMEM_SHARED((AGG_ROWS, HH), jnp.float32)]
            + [pltpu.SemaphoreType.DMA] * (2 * NBUF)
        ),
    )


def _sc_scatter(ma, mb, codep, bounds, zeros_hbm):
    return _sc_scatter_kernel()(ma, mb, codep, bounds, zeros_hbm)


_BM = 1000
_GRID = N // _BM


def _mm_body(h_ref, wa_ref, wb_ref, oa_ref, ob_ref):
    h = h_ref[...]
    oa_ref[...] = jnp.dot(h, wa_ref[...], preferred_element_type=jnp.float32)
    ob_ref[...] = jnp.dot(h, wb_ref[...], preferred_element_type=jnp.float32)


def _mm(h, wa, wb):
    return pl.pallas_call(
        _mm_body,
        grid=(_GRID,),
        in_specs=[
            pl.BlockSpec((_BM, H), lambda i: (i, 0)),
            pl.BlockSpec((H, HH), lambda i: (0, 0)),
            pl.BlockSpec((H, HH), lambda i: (0, 0)),
        ],
        out_specs=[
            pl.BlockSpec((_BM, HH), lambda i: (i, 0)),
            pl.BlockSpec((_BM, HH), lambda i: (i, 0)),
        ],
        out_shape=[
            jax.ShapeDtypeStruct((N, HH), jnp.float32),
            jax.ShapeDtypeStruct((N, HH), jnp.float32),
        ],
    )(h, wa, wb)


def _gru_math(aa, ab, h_ref,
              wira, wirb, wiza, wizb, wina, winb, whr, whz, whn,
              bir, biz, bin_, bhr, bhz, bhn):
    agga = aa[...]
    aggb = ab[...]
    h = h_ref[...]
    i_r = (jnp.dot(agga, wira[...], preferred_element_type=jnp.float32)
           + jnp.dot(aggb, wirb[...], preferred_element_type=jnp.float32)
           + bir[...])
    i_z = (jnp.dot(agga, wiza[...], preferred_element_type=jnp.float32)
           + jnp.dot(aggb, wizb[...], preferred_element_type=jnp.float32)
           + biz[...])
    i_n = (jnp.dot(agga, wina[...], preferred_element_type=jnp.float32)
           + jnp.dot(aggb, winb[...], preferred_element_type=jnp.float32)
           + bin_[...])
    h_r = jnp.dot(h, whr[...], preferred_element_type=jnp.float32) + bhr[...]
    h_z = jnp.dot(h, whz[...], preferred_element_type=jnp.float32) + bhz[...]
    h_n = jnp.dot(h, whn[...], preferred_element_type=jnp.float32) + bhn[...]
    r = jax.nn.sigmoid(i_r + h_r)
    z = jax.nn.sigmoid(i_z + h_z)
    n = jnp.tanh(i_n + r * h_n)
    return (1.0 - z) * n + z * h


def _gru_step_body(aa, ab, h_ref,
                   wira, wirb, wiza, wizb, wina, winb, whr, whz, whn,
                   bir, biz, bin_, bhr, bhz, bhn, wna, wnb,
                   h_out, ma_out, mb_out):
    hnew = _gru_math(aa, ab, h_ref,
                     wira, wirb, wiza, wizb, wina, winb, whr, whz, whn,
                     bir, biz, bin_, bhr, bhz, bhn)
    h_out[...] = hnew
    ma_out[...] = jnp.dot(hnew, wna[...], preferred_element_type=jnp.float32)
    mb_out[...] = jnp.dot(hnew, wnb[...], preferred_element_type=jnp.float32)


def _gru_pool_body(aa, ab, h_ref,
                   wira, wirb, wiza, wizb, wina, winb, whr, whz, whn,
                   bir, biz, bin_, bhr, bhz, bhn, pool_out):
    hnew = _gru_math(aa, ab, h_ref,
                     wira, wirb, wiza, wizb, wina, winb, whr, whz, whn,
                     bir, biz, bin_, bhr, bhz, bhn)

    @pl.when(pl.program_id(0) == 0)
    def _():
        pool_out[...] = jnp.zeros_like(pool_out)

    pool_out[...] += jnp.sum(hnew, axis=0, keepdims=True)


def _aspec():
    return pl.BlockSpec((_BM, HH), lambda i: (i, 0))


def _wspec():
    return pl.BlockSpec((HH, H), lambda i: (0, 0))


def _hwspec():
    return pl.BlockSpec((H, H), lambda i: (0, 0))


def _bspec():
    return pl.BlockSpec((1, H), lambda i: (0, 0))


_GRU_IN_SPECS = [
    _aspec(), _aspec(),
    pl.BlockSpec((_BM, H), lambda i: (i, 0)),
    _wspec(), _wspec(), _wspec(), _wspec(), _wspec(), _wspec(),
    _hwspec(), _hwspec(), _hwspec(),
    _bspec(), _bspec(), _bspec(),
    _bspec(), _bspec(), _bspec(),
]


def _gru_step(aa, ab, h, gw, wna, wnb):
    return pl.pallas_call(
        _gru_step_body,
        grid=(_GRID,),
        in_specs=_GRU_IN_SPECS + [pl.BlockSpec((H, HH), lambda i: (0, 0))] * 2,
        out_specs=[
            pl.BlockSpec((_BM, H), lambda i: (i, 0)),
            pl.BlockSpec((_BM, HH), lambda i: (i, 0)),
            pl.BlockSpec((_BM, HH), lambda i: (i, 0)),
        ],
        out_shape=[
            jax.ShapeDtypeStruct((N, H), jnp.float32),
            jax.ShapeDtypeStruct((N, HH), jnp.float32),
            jax.ShapeDtypeStruct((N, HH), jnp.float32),
        ],
    )(aa, ab, h, *gw, wna, wnb)


def _gru_pool(aa, ab, h, gw):
    return pl.pallas_call(
        _gru_pool_body,
        grid=(_GRID,),
        in_specs=_GRU_IN_SPECS,
        out_specs=pl.BlockSpec((1, H), lambda i: (0, 0)),
        out_shape=jax.ShapeDtypeStruct((1, H), jnp.float32),
    )(aa, ab, h, *gw)


def _mlp_body(p_ref, w1, b1, w2, b2, w3, b3, wc, bc, y_ref):
    o = jnp.maximum(jnp.dot(p_ref[...], w1[...],
                            preferred_element_type=jnp.float32) + b1[...], 0.0)
    o = jnp.maximum(jnp.dot(o, w2[...],
                            preferred_element_type=jnp.float32) + b2[...], 0.0)
    o = jnp.maximum(jnp.dot(o, w3[...],
                            preferred_element_type=jnp.float32) + b3[...], 0.0)
    y_ref[...] = jax.nn.sigmoid(
        jnp.dot(o, wc[...], preferred_element_type=jnp.float32) + bc[...])


def _mlp(pooled, w1, b1, w2, b2, w3, b3, wc, bc):
    return pl.pallas_call(
        _mlp_body,
        out_shape=jax.ShapeDtypeStruct((1, 2), jnp.float32),
    )(pooled, w1, b1, w2, b2, w3, b3, wc, bc)


def kernel(x, edge_index, ggnn_weight, gru_w_ih, gru_w_hh, gru_b_ih, gru_b_hh,
           ef_l1_w, ef_l1_b, ef_f1_w, ef_f1_b, ef_f2_w, ef_f2_b, cls_w, cls_b):
    ei = edge_index.astype(jnp.int32)
    src = ei[0]
    dst = ei[1]
    qb = jnp.asarray(QB, jnp.int32)
    q = ((dst >= QB[1]).astype(jnp.int32) + (dst >= QB[2])
         + (dst >= QB[3]))
    code = (src << 12) | (dst - jnp.take(qb, q))
    sq, scode = lax.sort((q, code), num_keys=1, is_stable=False)
    cbound = jnp.searchsorted(
        sq, jnp.arange(1, 5, dtype=jnp.int32)).astype(jnp.int32)
    cstart = jnp.concatenate([jnp.zeros((1,), jnp.int32), cbound[:3]])
    cnts = cbound - cstart
    tchunks = (cnts + K - 1) // K
    bstart = jnp.concatenate(
        [jnp.zeros((1,), jnp.int32), jnp.cumsum(tchunks)])
    slot = jnp.arange(ECAP, dtype=jnp.int32)
    qslot = ((slot >= bstart[1] * K).astype(jnp.int32)
             + (slot >= bstart[2] * K) + (slot >= bstart[3] * K))
    off = slot - jnp.take(bstart, qslot) * K
    sidx = jnp.take(cstart, qslot) + off
    codep = jnp.where(
        off < jnp.take(cnts, qslot),
        jnp.take(scode, jnp.minimum(sidx, E - 1)), DUMMY)
    codep = codep.reshape(CT, NS, K).swapaxes(0, 1)
    bounds = jnp.zeros((16,), jnp.int32)
    for _j in (0, 1):
        for _c in (0, 1):
            _q = 2 * _j + _c
            bounds = (bounds.at[4 * _j + 2 * _c].set(bstart[_q])
                      .at[4 * _j + 2 * _c + 1].set(bstart[_q + 1])
                      .at[8 + 2 * _j + _c].set(QB[_q]))
    zeros_hbm = jnp.zeros((ROWS_PER_TILE, HH), jnp.float32)

    h0 = jnp.pad(x, ((0, 0), (0, H - x.shape[1])))
    wg = jnp.pad(ggnn_weight, ((0, 0), (0, 0), (0, 2 * HH - H)))
    wga = wg[:, :, :HH]
    wgb = wg[:, :, HH:]
    wihT = jnp.pad(gru_w_ih.T, ((0, 2 * HH - H), (0, 0)))
    whhT = gru_w_hh.T
    gw = (
        wihT[:HH, 0:H], wihT[HH:, 0:H],
        wihT[:HH, H:2 * H], wihT[HH:, H:2 * H],
        wihT[:HH, 2 * H:], wihT[HH:, 2 * H:],
        whhT[:, 0:H], whhT[:, H:2 * H], whhT[:, 2 * H:3 * H],
        gru_b_ih[0:H].reshape(1, H), gru_b_ih[H:2 * H].reshape(1, H),
        gru_b_ih[2 * H:].reshape(1, H),
        gru_b_hh[0:H].reshape(1, H), gru_b_hh[H:2 * H].reshape(1, H),
        gru_b_hh[2 * H:].reshape(1, H),
    )

    h = h0
    ma, mb = _mm(h0, wga[0], wgb[0])
    for i in range(NUM_STEPS):
        agg_a, agg_b = _sc_scatter(ma, mb, codep, bounds, zeros_hbm)
        if i < NUM_STEPS - 1:
            h, ma, mb = _gru_step(agg_a, agg_b, h, gw, wga[i + 1], wgb[i + 1])
        else:
            pooled = _gru_pool(agg_a, agg_b, h, gw)

    return _mlp(pooled,
                ef_l1_w.T, ef_l1_b.reshape(1, -1),
                ef_f1_w.T, ef_f1_b.reshape(1, -1),
                ef_f2_w.T, ef_f2_b.reshape(1, -1),
                cls_w.T, cls_b.reshape(1, -1))
